# baseline (speedup 1.0000x reference)
import jax
import jax.numpy as jnp
from jax.experimental import pallas as pl
from jax.experimental.pallas import tpu as pltpu

MARGIN = 0.3
PD_EPS = 1e-6
B = 1024
D = 2048
BM = 256
NI = B // BM
PC = 256
NC = B // PC


def _triplet_kernel(x1_ref, x2_ref, out_ref,
                    pbf_ref, colp_ref, anbf_ref, hmin_ref, hpos_ref,
                    araw_ref, praw_ref, asem, psem):
    i = pl.program_id(0)
    c = pl.program_id(1)
    ones_row = jnp.ones((1, D), jnp.float32)

    @pl.when(jnp.logical_and(i == 0, c == 0))
    def _start_dmas():
        pltpu.make_async_copy(
            x1_ref.at[pl.ds(0, BM), 0, :],
            araw_ref.at[0], asem.at[0]).start()
        for k in range(NC):
            pltpu.make_async_copy(
                x2_ref.at[pl.ds(k * PC, PC), 1, :],
                praw_ref.at[k], psem.at[k]).start()
        for k in range(1, NI):
            pltpu.make_async_copy(
                x1_ref.at[pl.ds(k * BM, BM), 0, :],
                araw_ref.at[k], asem.at[k]).start()
        out_ref[...] = jnp.zeros_like(out_ref)

    @pl.when(i == 0)
    def _prep_chunk():
        pltpu.make_async_copy(
            x2_ref.at[pl.ds(c * PC, PC), 1, :],
            praw_ref.at[c], psem.at[c]).wait()
        praw = praw_ref[c]
        np2 = jax.lax.dot_general(
            praw * praw, ones_row, (((1,), (1,)), ((), ())),
            preferred_element_type=jnp.float32)
        sump = jax.lax.dot_general(
            praw, ones_row, (((1,), (1,)), ((), ())),
            preferred_element_type=jnp.float32)
        t = 1.0 / jnp.maximum(jnp.sqrt(np2), 1e-12)
        pbf_ref[pl.ds(c * PC, PC), :] = (praw * (2.0 * t)).astype(jnp.bfloat16)
        colp_ref[pl.ds(c * PC, PC), :] = np2 * t * t - (2.0 * PD_EPS) * sump * t

    @pl.when(c == 0)
    def _prep_anchor():
        pltpu.make_async_copy(
            x1_ref.at[pl.ds(i * BM, BM), 0, :],
            araw_ref.at[i], asem.at[i]).wait()
        a = araw_ref[i]
        na2 = jax.lax.dot_general(
            a * a, ones_row, (((1,), (1,)), ((), ())),
            preferred_element_type=jnp.float32)
        ta = 1.0 / jnp.maximum(jnp.sqrt(na2), 1e-12)
        anbf_ref[...] = (a * ta).astype(jnp.bfloat16)

    cross = jax.lax.dot_general(
        pbf_ref[pl.ds(c * PC, PC), :], anbf_ref[...],
        (((1,), (1,)), ((), ())),
        preferred_element_type=jnp.float32)
    h = colp_ref[pl.ds(c * PC, PC), :] - cross

    @pl.when(c == i)
    def _diag_chunk():
        rowj = jax.lax.broadcasted_iota(jnp.int32, (PC, BM), 0)
        coli = jax.lax.broadcasted_iota(jnp.int32, (PC, BM), 1)
        eye = rowj == coli
        hpos_ref[...] = jnp.sum(jnp.where(eye, h, 0.0), axis=0, keepdims=True)
        m = jnp.min(jnp.where(eye, jnp.float32(3.0e38), h), axis=0,
                    keepdims=True)
        hmin_ref[...] = jnp.where(c == 0, m, jnp.minimum(hmin_ref[...], m))

    @pl.when(c != i)
    def _off_chunk():
        m = jnp.min(h, axis=0, keepdims=True)
        hmin_ref[...] = jnp.where(c == 0, m, jnp.minimum(hmin_ref[...], m))

    @pl.when(c == NC - 1)
    def _finish_block():
        lv = jnp.maximum(hpos_ref[...] - hmin_ref[...] + MARGIN, 0.0) * (1.0 / B)
        out_ref[...] += jnp.sum(lv, axis=1, keepdims=True)


def kernel(x1, x2):
    out = pl.pallas_call(
        _triplet_kernel,
        grid=(NI, NC),
        in_specs=[
            pl.BlockSpec(memory_space=pltpu.HBM),
            pl.BlockSpec(memory_space=pltpu.HBM),
        ],
        out_specs=pl.BlockSpec((1, 1), lambda i, c: (0, 0)),
        out_shape=jax.ShapeDtypeStruct((1, 1), jnp.float32),
        scratch_shapes=[
            pltpu.VMEM((B, D), jnp.bfloat16),
            pltpu.VMEM((B, 1), jnp.float32),
            pltpu.VMEM((BM, D), jnp.bfloat16),
            pltpu.VMEM((1, BM), jnp.float32),
            pltpu.VMEM((1, BM), jnp.float32),
            pltpu.VMEM((NI, BM, D), jnp.float32),
            pltpu.VMEM((NC, PC, D), jnp.float32),
            pltpu.SemaphoreType.DMA((NI,)),
            pltpu.SemaphoreType.DMA((NC,)),
        ],
        compiler_params=pltpu.CompilerParams(
            dimension_semantics=("arbitrary", "arbitrary"),
        ),
    )(x1, x2)
    return out[0, 0]


# speedup vs baseline: 1.0608x; 1.0608x over previous
import jax
import jax.numpy as jnp
from jax.experimental import pallas as pl
from jax.experimental.pallas import tpu as pltpu

MARGIN = 0.3
PD_EPS = 1e-6
B = 1024
D = 2048
BM = 256
NI = B // BM
PC = 256
NC = B // PC
BIG = 3.0e38


def _triplet_kernel(x1_ref, x2_ref, out_ref,
                    pbf_ref, colp_ref, araw_ref, praw_ref, asem, psem):
    ones_row = jnp.ones((1, D), jnp.float32)

    pltpu.make_async_copy(
        x1_ref.at[pl.ds(0, BM), 0, :],
        araw_ref.at[0], asem.at[0]).start()
    for k in range(NC):
        pltpu.make_async_copy(
            x2_ref.at[pl.ds(k * PC, PC), 1, :],
            praw_ref.at[k], psem.at[k]).start()
    for k in range(1, NI):
        pltpu.make_async_copy(
            x1_ref.at[pl.ds(k * BM, BM), 0, :],
            araw_ref.at[k], asem.at[k]).start()

    for c in range(NC):
        pltpu.make_async_copy(
            x2_ref.at[pl.ds(c * PC, PC), 1, :],
            praw_ref.at[c], psem.at[c]).wait()
        praw = praw_ref[c]
        np2 = jax.lax.dot_general(
            praw * praw, ones_row, (((1,), (1,)), ((), ())),
            preferred_element_type=jnp.float32)
        sump = jax.lax.dot_general(
            praw, ones_row, (((1,), (1,)), ((), ())),
            preferred_element_type=jnp.float32)
        t = 1.0 / jnp.maximum(jnp.sqrt(np2), 1e-12)
        pbf_ref[c * PC:(c + 1) * PC, :] = (praw * (2.0 * t)).astype(jnp.bfloat16)
        colp_ref[c * PC:(c + 1) * PC, :] = np2 * t * t - (2.0 * PD_EPS) * sump * t

    eye = (jax.lax.broadcasted_iota(jnp.int32, (BM, BM), 0)
           == jax.lax.broadcasted_iota(jnp.int32, (BM, BM), 1))

    total = jnp.zeros((1, 1), jnp.float32)
    for i in range(NI):
        pltpu.make_async_copy(
            x1_ref.at[pl.ds(i * BM, BM), 0, :],
            araw_ref.at[i], asem.at[i]).wait()
        a = araw_ref[i]
        na2 = jax.lax.dot_general(
            a * a, ones_row, (((1,), (1,)), ((), ())),
            preferred_element_type=jnp.float32)
        ta = 1.0 / jnp.maximum(jnp.sqrt(na2), 1e-12)
        anbf = (a * ta).astype(jnp.bfloat16)

        cross = jax.lax.dot_general(
            pbf_ref[...], anbf, (((1,), (1,)), ((), ())),
            preferred_element_type=jnp.float32)
        h = colp_ref[...] - cross

        hsub = h[i * BM:(i + 1) * BM, :]
        hpos = jnp.sum(jnp.where(eye, hsub, 0.0), axis=0, keepdims=True)
        hmin = jnp.min(jnp.where(eye, BIG, hsub), axis=0, keepdims=True)
        if i > 0:
            hmin = jnp.minimum(hmin, jnp.min(h[:i * BM, :], axis=0,
                                             keepdims=True))
        if i < NI - 1:
            hmin = jnp.minimum(hmin, jnp.min(h[(i + 1) * BM:, :], axis=0,
                                             keepdims=True))

        lv = jnp.maximum(hpos - hmin + MARGIN, 0.0)
        total = total + jnp.sum(lv, axis=1, keepdims=True)

    out_ref[...] = total * (1.0 / B)


def kernel(x1, x2):
    out = pl.pallas_call(
        _triplet_kernel,
        in_specs=[
            pl.BlockSpec(memory_space=pltpu.HBM),
            pl.BlockSpec(memory_space=pltpu.HBM),
        ],
        out_shape=jax.ShapeDtypeStruct((1, 1), jnp.float32),
        scratch_shapes=[
            pltpu.VMEM((B, D), jnp.bfloat16),
            pltpu.VMEM((B, 1), jnp.float32),
            pltpu.VMEM((NI, BM, D), jnp.float32),
            pltpu.VMEM((NC, PC, D), jnp.float32),
            pltpu.SemaphoreType.DMA((NI,)),
            pltpu.SemaphoreType.DMA((NC,)),
        ],
    )(x1, x2)
    return out[0, 0]


# speedup vs baseline: 1.4085x; 1.3278x over previous
import jax
import jax.numpy as jnp
from jax.experimental import pallas as pl
from jax.experimental.pallas import tpu as pltpu

MARGIN = 0.3
PD_EPS = 1e-6
B = 1024
D = 2048
BM = 256
NI = B // BM
PC = 256
NC = B // PC
BIG = 3.0e38


def _triplet_kernel(x1_ref, x2_ref, out_ref,
                    pbf_ref, colp_ref, araw_ref, praw_ref, asem, psem):
    i = pl.program_id(0)
    ones_row = jnp.ones((1, D), jnp.float32)

    @pl.when(i == 0)
    def _init():
        pltpu.make_async_copy(
            x1_ref.at[pl.ds(0, BM), 0, :],
            araw_ref.at[0], asem.at[0]).start()
        for k in range(NC):
            pltpu.make_async_copy(
                x2_ref.at[pl.ds(k * PC, PC), 1, :],
                praw_ref.at[k], psem.at[k]).start()
        for k in range(1, NI):
            pltpu.make_async_copy(
                x1_ref.at[pl.ds(k * BM, BM), 0, :],
                araw_ref.at[k], asem.at[k]).start()
        out_ref[...] = jnp.zeros_like(out_ref)
        for c in range(NC):
            pltpu.make_async_copy(
                x2_ref.at[pl.ds(c * PC, PC), 1, :],
                praw_ref.at[c], psem.at[c]).wait()
            praw = praw_ref[c]
            np2 = jax.lax.dot_general(
                praw * praw, ones_row, (((1,), (1,)), ((), ())),
                preferred_element_type=jnp.float32)
            sump = jax.lax.dot_general(
                praw, ones_row, (((1,), (1,)), ((), ())),
                preferred_element_type=jnp.float32)
            t = 1.0 / jnp.maximum(jnp.sqrt(np2), 1e-12)
            pbf_ref[c * PC:(c + 1) * PC, :] = (praw * (2.0 * t)).astype(jnp.bfloat16)
            colp_ref[c * PC:(c + 1) * PC, :] = np2 * t * t - (2.0 * PD_EPS) * sump * t

    pltpu.make_async_copy(
        x1_ref.at[pl.ds(i * BM, BM), 0, :],
        araw_ref.at[i], asem.at[i]).wait()
    a = araw_ref[i]
    na2 = jax.lax.dot_general(
        a * a, ones_row, (((1,), (1,)), ((), ())),
        preferred_element_type=jnp.float32)
    ta = 1.0 / jnp.maximum(jnp.sqrt(na2), 1e-12)
    anbf = (a * ta).astype(jnp.bfloat16)

    cross = jax.lax.dot_general(
        pbf_ref[...], anbf, (((1,), (1,)), ((), ())),
        preferred_element_type=jnp.float32)
    h = colp_ref[...] - cross

    eye = (jax.lax.broadcasted_iota(jnp.int32, (BM, BM), 0)
           == jax.lax.broadcasted_iota(jnp.int32, (BM, BM), 1))

    hmin = jnp.full((1, BM), BIG, jnp.float32)
    hpos = jnp.zeros((1, BM), jnp.float32)
    for k in range(NC):
        chunk = h[k * BM:(k + 1) * BM, :]
        is_diag = k == i
        m_plain = jnp.min(chunk, axis=0, keepdims=True)
        m_mask = jnp.min(jnp.where(eye, BIG, chunk), axis=0, keepdims=True)
        s_diag = jnp.sum(jnp.where(eye, chunk, 0.0), axis=0, keepdims=True)
        hmin = jnp.minimum(hmin, jnp.where(is_diag, m_mask, m_plain))
        hpos = hpos + jnp.where(is_diag, s_diag, 0.0)

    lv = jnp.maximum(hpos - hmin + MARGIN, 0.0)
    out_ref[...] += jnp.sum(lv, axis=1, keepdims=True) * (1.0 / B)


def kernel(x1, x2):
    out = pl.pallas_call(
        _triplet_kernel,
        grid=(NI,),
        in_specs=[
            pl.BlockSpec(memory_space=pltpu.HBM),
            pl.BlockSpec(memory_space=pltpu.HBM),
        ],
        out_specs=pl.BlockSpec((1, 1), lambda i: (0, 0)),
        out_shape=jax.ShapeDtypeStruct((1, 1), jnp.float32),
        scratch_shapes=[
            pltpu.VMEM((B, D), jnp.bfloat16),
            pltpu.VMEM((B, 1), jnp.float32),
            pltpu.VMEM((NI, BM, D), jnp.float32),
            pltpu.VMEM((NC, PC, D), jnp.float32),
            pltpu.SemaphoreType.DMA((NI,)),
            pltpu.SemaphoreType.DMA((NC,)),
        ],
        compiler_params=pltpu.CompilerParams(
            dimension_semantics=("arbitrary",),
        ),
    )(x1, x2)
    return out[0, 0]
